# baseline (speedup 1.0000x reference)
import jax
import jax.numpy as jnp
from jax.experimental import pallas as pl

_ES = [8, 16, 64, 256, 1024]
_HID = 256
_B = 16
_N = 1024
_LAT = 64
_OUT = 3

def _mm(a, b):
    return jax.lax.dot_general(
        a.astype(jnp.bfloat16), b.astype(jnp.bfloat16),
        (((1,), (0,)), ((), ())),
        preferred_element_type=jnp.float32)


def _small_body(lat_ref, wg0, wg1, wg2, wg3, wg4, bg0, bg1, bg2, bg3, bg4,
                w0r, bp0, bp1, bp2, bp3, bp4,
                g1o, g2o, g3o, g4o, cw0o, cb0o, cb1o, cb2o, cb3o, cb4o):
    wgs = (wg0, wg1, wg2, wg3, wg4)
    bgs = (bg0, bg1, bg2, bg3, bg4)
    bps = (bp0, bp1, bp2, bp3, bp4)
    gouts = (None, g1o, g2o, g3o, g4o)
    cbouts = (cb0o, cb1o, cb2o, cb3o, cb4o)
    for l in range(5):
        lat = lat_ref[:, l, :]
        g = _mm(lat, wgs[l][...]) + bgs[l][...]
        if l == 0:
            cw0o[...] = _mm(g, w0r[...])
        else:
            gouts[l][...] = g
        cbouts[l][...] = _mm(g, bps[l][...])


def _small_call(latents, wgs, bgs, w0r, bps):
    outs = [
        jax.ShapeDtypeStruct((_B, _ES[1]), jnp.float32),
        jax.ShapeDtypeStruct((_B, _ES[2]), jnp.float32),
        jax.ShapeDtypeStruct((_B, _ES[3]), jnp.float32),
        jax.ShapeDtypeStruct((_B, _ES[4]), jnp.float32),
        jax.ShapeDtypeStruct((_B, _HID * 2), jnp.float32),
        jax.ShapeDtypeStruct((_B, _HID), jnp.float32),
        jax.ShapeDtypeStruct((_B, _HID), jnp.float32),
        jax.ShapeDtypeStruct((_B, _HID), jnp.float32),
        jax.ShapeDtypeStruct((_B, _HID), jnp.float32),
        jax.ShapeDtypeStruct((_B, _OUT), jnp.float32),
    ]
    return pl.pallas_call(_small_body, out_shape=outs)(
        latents, *wgs, *bgs, w0r, *bps)


def _cw_body(g_ref, w_ref, o_ref):
    o_ref[...] = _mm(g_ref[...], w_ref[...])


def _cw_call(g, wr, fc):
    e = wr.shape[0]
    f = wr.shape[1]
    nc = f // fc
    return pl.pallas_call(
        _cw_body,
        grid=(nc,),
        in_specs=[
            pl.BlockSpec((_B, e), lambda c: (0, 0)),
            pl.BlockSpec((e, fc), lambda c: (0, c)),
        ],
        out_specs=pl.BlockSpec((_B, fc), lambda c: (0, c)),
        out_shape=jax.ShapeDtypeStruct((_B, f), jnp.float32),
    )(g, wr)


def _apply_body(ct_ref, cw0, cw1, cw2, cw3, cw4,
                cb0, cb1, cb2, cb3, cb4, o_ref):
    x = ct_ref[0]
    h = _mm(cw0[0], x) + cb0[0]
    x = jnp.sin(30.0 * h)
    for cw, cb in ((cw1, cb1), (cw2, cb2), (cw3, cb3)):
        h = _mm(cw[0], x) + cb[0]
        x = jnp.sin(30.0 * h)
    o_ref[0] = _mm(cw4[0], x) + cb4[0]


def _apply_call(coords_t, cws, cbs):
    in_specs = [pl.BlockSpec((1, 2, _N), lambda b: (b, 0, 0))]
    for cw in cws:
        in_specs.append(
            pl.BlockSpec((1,) + cw.shape[1:], lambda b: (b, 0, 0)))
    for cb in cbs:
        in_specs.append(
            pl.BlockSpec((1, cb.shape[1], 1), lambda b: (b, 0, 0)))
    return pl.pallas_call(
        _apply_body,
        grid=(_B,),
        in_specs=in_specs,
        out_specs=pl.BlockSpec((1, _OUT, _N), lambda b: (b, 0, 0)),
        out_shape=jax.ShapeDtypeStruct((_B, _OUT, _N), jnp.float32),
    )(coords_t, *cws, *cbs)


def kernel(latents, coords, Wg0, Wg1, Wg2, Wg3, Wg4,
           bg0, bg1, bg2, bg3, bg4,
           W0, W1, W2, W3, W4, b0, b1, b2, b3, b4):
    wgs = tuple(w.T for w in (Wg0, Wg1, Wg2, Wg3, Wg4))
    bgs = tuple(b.reshape(1, -1) for b in (bg0, bg1, bg2, bg3, bg4))
    outs = [_HID, _HID, _HID, _HID, _OUT]
    bps = tuple(b.reshape(e, o) for b, e, o in
                zip((b0, b1, b2, b3, b4), _ES, outs))
    w0r = W0.reshape(_ES[0], _HID * 2)

    (g1, g2, g3, g4, cw0f, cb0, cb1, cb2, cb3, cb4) = _small_call(
        latents, wgs, bgs, w0r, bps)

    cw1f = _cw_call(g1, W1.reshape(_ES[1], _HID * _HID), 16384)
    cw2f = _cw_call(g2, W2.reshape(_ES[2], _HID * _HID), 16384)
    cw3f = _cw_call(g3, W3.reshape(_ES[3], _HID * _HID), 8192)
    cw4f = _cw_call(g4, W4.reshape(_ES[4], _OUT * _HID), _OUT * _HID)

    cws = (cw0f.reshape(_B, _HID, 2),
           cw1f.reshape(_B, _HID, _HID),
           cw2f.reshape(_B, _HID, _HID),
           cw3f.reshape(_B, _HID, _HID),
           cw4f.reshape(_B, _OUT, _HID))
    cbs = tuple(cb.reshape(_B, -1, 1) for cb in (cb0, cb1, cb2, cb3, cb4))

    out_t = _apply_call(coords.transpose(0, 2, 1), cws, cbs)
    return out_t.transpose(0, 2, 1)


# speedup vs baseline: 1.0792x; 1.0792x over previous
import jax
import jax.numpy as jnp
from jax.experimental import pallas as pl
from jax.experimental.pallas import tpu as pltpu

_ES = [8, 16, 64, 256, 1024]
_HID = 256
_B = 16
_N = 1024
_LAT = 64
_OUT = 3


def _mm(a, b):
    return jax.lax.dot_general(
        a.astype(jnp.bfloat16), b.astype(jnp.bfloat16),
        (((1,), (0,)), ((), ())),
        preferred_element_type=jnp.float32)


def _small_body(lat_ref, wg0, wg1, wg2, wg3, wg4, bg0, bg1, bg2, bg3, bg4,
                w0r, bp0, bp1, bp2, bp3, bp4,
                g1o, g2o, g3o, g4o, cw0o, cb0o, cb1o, cb2o, cb3o, cb4o):
    wgs = (wg0, wg1, wg2, wg3, wg4)
    bgs = (bg0, bg1, bg2, bg3, bg4)
    bps = (bp0, bp1, bp2, bp3, bp4)
    gouts = (None, g1o, g2o, g3o, g4o)
    cbouts = (cb0o, cb1o, cb2o, cb3o, cb4o)
    for l in range(5):
        lat = lat_ref[:, l, :]
        g = _mm(lat, wgs[l][...]) + bgs[l][...]
        if l == 0:
            cw0o[...] = _mm(g, w0r[...])
        else:
            gouts[l][...] = g
        cbouts[l][...] = _mm(g, bps[l][...])


def _small_call(latents, wgs, bgs, w0r, bps):
    outs = [
        jax.ShapeDtypeStruct((_B, _ES[1]), jnp.float32),
        jax.ShapeDtypeStruct((_B, _ES[2]), jnp.float32),
        jax.ShapeDtypeStruct((_B, _ES[3]), jnp.float32),
        jax.ShapeDtypeStruct((_B, _ES[4]), jnp.float32),
        jax.ShapeDtypeStruct((_B, _HID * 2), jnp.float32),
        jax.ShapeDtypeStruct((_B, _HID), jnp.float32),
        jax.ShapeDtypeStruct((_B, _HID), jnp.float32),
        jax.ShapeDtypeStruct((_B, _HID), jnp.float32),
        jax.ShapeDtypeStruct((_B, _HID), jnp.float32),
        jax.ShapeDtypeStruct((_B, _OUT), jnp.float32),
    ]
    return pl.pallas_call(_small_body, out_shape=outs)(
        latents, *wgs, *bgs, w0r, *bps)


def _mega_body(ct, cw0, g1, g2, g3, g4, cb0, cbs, cb4,
               w1, w2, w3, w4, o_ref, X, C, C4a, C4b, C4c):
    C4s = (C4a, C4b, C4c)
    s = pl.program_id(0)
    l = s // _B
    b = s % _B

    @pl.when(s < 4)
    def _():
        v = _mm(g1[...], w1[...])
        C[0, :, pl.ds(s * 64, 64), :] = (
            v.reshape(_B, 64, _HID).astype(jnp.bfloat16))

    @pl.when(s < 16)
    def _():
        v = _mm(g2[...], w2[...])
        C[1, :, pl.ds(s * 16, 16), :] = (
            v.reshape(_B, 16, _HID).astype(jnp.bfloat16))

    @pl.when((s >= 8) & (s < 40) & ((s - 8) % 2 == 0))
    def _():
        c = (s - 8) // 2
        v = _mm(g3[...], w3[...])
        C[2, :, pl.ds(c * 16, 16), :] = (
            v.reshape(_B, 16, _HID).astype(jnp.bfloat16))

    for c, C4c in enumerate(C4s):
        @pl.when(s == 40 + c)
        def _(c=c, C4c=C4c):
            v = _mm(g4[...], w4[...])
            C4c[...] = v.reshape(_B, 1, _HID).astype(jnp.bfloat16)

    @pl.when(l == 0)
    def _():
        h = _mm(cw0[b], ct[0]) + cb0[b]
        X[b] = jnp.sin(30.0 * h).astype(jnp.bfloat16)

    @pl.when((l >= 1) & (l <= 3))
    def _():
        lc = l - 1
        h = _mm(C[lc, b], X[b]) + cbs[lc, b]
        X[b] = jnp.sin(30.0 * h).astype(jnp.bfloat16)

    @pl.when(l == 4)
    def _():
        cw4 = jnp.concatenate([c4[b] for c4 in C4s], axis=0)
        o_ref[0] = _mm(cw4, X[b]) + cb4[b]


def _mega_call(coords_t, cw0, gs, cb0, cbs, cb4, w1r, w2r, w3r, w4r):
    g1, g2, g3, g4 = gs
    in_specs = [
        pl.BlockSpec((1, 2, _N), lambda s: (jnp.clip(s, 0, 15), 0, 0)),
        pl.BlockSpec((_B, _HID, 2), lambda s: (0, 0, 0)),
        pl.BlockSpec(g1.shape, lambda s: (0, 0)),
        pl.BlockSpec(g2.shape, lambda s: (0, 0)),
        pl.BlockSpec(g3.shape, lambda s: (0, 0)),
        pl.BlockSpec(g4.shape, lambda s: (0, 0)),
        pl.BlockSpec(cb0.shape, lambda s: (0, 0, 0)),
        pl.BlockSpec(cbs.shape, lambda s: (0, 0, 0, 0)),
        pl.BlockSpec(cb4.shape, lambda s: (0, 0, 0)),
        pl.BlockSpec((_ES[1], 16384), lambda s: (0, jnp.clip(s, 0, 3))),
        pl.BlockSpec((_ES[2], 4096), lambda s: (0, jnp.clip(s, 0, 15))),
        pl.BlockSpec((_ES[3], 4096),
                     lambda s: (0, jnp.clip((s - 8) // 2, 0, 15))),
        pl.BlockSpec((_ES[4], _HID), lambda s: (0, jnp.clip(s - 40, 0, 2))),
    ]
    return pl.pallas_call(
        _mega_body,
        grid=(5 * _B,),
        in_specs=in_specs,
        out_specs=pl.BlockSpec((1, _OUT, _N),
                               lambda s: (jnp.clip(s - 64, 0, 15), 0, 0)),
        out_shape=jax.ShapeDtypeStruct((_B, _OUT, _N), jnp.float32),
        scratch_shapes=[
            pltpu.VMEM((_B, _HID, _N), jnp.bfloat16),
            pltpu.VMEM((3, _B, _HID, _HID), jnp.bfloat16),
            pltpu.VMEM((_B, 1, _HID), jnp.bfloat16),
            pltpu.VMEM((_B, 1, _HID), jnp.bfloat16),
            pltpu.VMEM((_B, 1, _HID), jnp.bfloat16),
        ],
    )(coords_t, cw0, g1, g2, g3, g4, cb0, cbs, cb4, w1r, w2r, w3r, w4r)


def kernel(latents, coords, Wg0, Wg1, Wg2, Wg3, Wg4,
           bg0, bg1, bg2, bg3, bg4,
           W0, W1, W2, W3, W4, b0, b1, b2, b3, b4):
    wgs = tuple(w.T for w in (Wg0, Wg1, Wg2, Wg3, Wg4))
    bgs = tuple(b.reshape(1, -1) for b in (bg0, bg1, bg2, bg3, bg4))
    outs = [_HID, _HID, _HID, _HID, _OUT]
    bps = tuple(b.reshape(e, o) for b, e, o in
                zip((b0, b1, b2, b3, b4), _ES, outs))
    w0r = W0.reshape(_ES[0], _HID * 2)

    (g1, g2, g3, g4, cw0f, cb0, cb1, cb2, cb3, cb4) = _small_call(
        latents, wgs, bgs, w0r, bps)

    out_t = _mega_call(
        coords.transpose(0, 2, 1),
        cw0f.reshape(_B, _HID, 2),
        (g1, g2, g3, g4),
        cb0.reshape(_B, _HID, 1),
        jnp.stack([cb1, cb2, cb3]).reshape(3, _B, _HID, 1),
        cb4.reshape(_B, _OUT, 1),
        W1.reshape(_ES[1], _HID * _HID),
        W2.reshape(_ES[2], _HID * _HID),
        W3.reshape(_ES[3], _HID * _HID),
        W4.reshape(_ES[4], _OUT * _HID),
    )
    return out_t.transpose(0, 2, 1)


# speedup vs baseline: 1.0894x; 1.0094x over previous
import jax
import jax.numpy as jnp
from jax.experimental import pallas as pl
from jax.experimental.pallas import tpu as pltpu

_ES = [8, 16, 64, 256, 1024]
_HID = 256
_B = 16
_N = 1024
_LAT = 64
_OUT = 3


def _mm(a, b):
    return jax.lax.dot_general(
        a.astype(jnp.bfloat16), b.astype(jnp.bfloat16),
        (((1,), (0,)), ((), ())),
        preferred_element_type=jnp.float32)


def _small_body(lat_ref, wg0, wg1, wg2, wg3, wg4, bg0, bg1, bg2, bg3, bg4,
                w0r, bp0, bp1, bp2, bp3, bp4,
                g1o, g2o, g3o, g4o, cw0o, cb0o, cb1o, cb2o, cb3o, cb4o):
    wgs = (wg0, wg1, wg2, wg3, wg4)
    bgs = (bg0, bg1, bg2, bg3, bg4)
    bps = (bp0, bp1, bp2, bp3, bp4)
    gouts = (None, g1o, g2o, g3o, g4o)
    cbouts = (cb0o, cb1o, cb2o, cb3o, cb4o)
    for l in range(5):
        lat = lat_ref[:, l, :]
        g = _mm(lat, wgs[l][...]) + bgs[l][...]
        if l == 0:
            cw0o[...] = _mm(g, w0r[...])
        else:
            gouts[l][...] = g
        cbouts[l][...] = _mm(g, bps[l][...])


def _small_call(latents, wgs, bgs, w0r, bps):
    outs = [
        jax.ShapeDtypeStruct((_B, _ES[1]), jnp.float32),
        jax.ShapeDtypeStruct((_B, _ES[2]), jnp.float32),
        jax.ShapeDtypeStruct((_B, _ES[3]), jnp.float32),
        jax.ShapeDtypeStruct((_B, _ES[4]), jnp.float32),
        jax.ShapeDtypeStruct((_B, _HID * 2), jnp.float32),
        jax.ShapeDtypeStruct((_B, _HID), jnp.float32),
        jax.ShapeDtypeStruct((_B, _HID), jnp.float32),
        jax.ShapeDtypeStruct((_B, _HID), jnp.float32),
        jax.ShapeDtypeStruct((_B, _HID), jnp.float32),
        jax.ShapeDtypeStruct((_B, _OUT), jnp.float32),
    ]
    return pl.pallas_call(_small_body, out_shape=outs)(
        latents, *wgs, *bgs, w0r, *bps)


def _mega_body(ct, cw0, g1, g2, g3, g4, cbs, cb4,
               w1, w2, w3, w4, o_ref, X, C, C4a, C4b, C4c):
    C4s = (C4a, C4b, C4c)
    s = pl.program_id(0)
    l = s // _B
    b = s % _B

    @pl.when(s < 4)
    def _():
        v = _mm(g1[...], w1[...])
        C[0, :, pl.ds(s * 64, 64), :] = (
            v.reshape(_B, 64, _HID).astype(jnp.bfloat16))

    @pl.when(s < 16)
    def _():
        v = _mm(g2[...], w2[...])
        C[1, :, pl.ds(s * 16, 16), :] = (
            v.reshape(_B, 16, _HID).astype(jnp.bfloat16))

    @pl.when((s >= 8) & (s < 40) & ((s - 8) % 2 == 0))
    def _():
        c = (s - 8) // 2
        v = _mm(g3[...], w3[...])
        C[2, :, pl.ds(c * 16, 16), :] = (
            v.reshape(_B, 16, _HID).astype(jnp.bfloat16))

    for c, C4c in enumerate(C4s):
        @pl.when(s == 40 + c)
        def _(c=c, C4c=C4c):
            v = _mm(g4[...], w4[...])
            C4c[...] = v.reshape(_B, 1, _HID).astype(jnp.bfloat16)

    @pl.when(s == 0)
    def _():
        C[3] = jnp.concatenate(
            [cw0[...].astype(jnp.bfloat16),
             jnp.zeros((_B, _HID, _HID - 2), jnp.bfloat16)], axis=2)

    @pl.when(l == 0)
    def _():
        X[b] = jnp.concatenate(
            [ct[0].astype(jnp.bfloat16),
             jnp.zeros((_HID - 2, _N), jnp.bfloat16)], axis=0)

    lc = jnp.where(l == 0, 3, l - 1)
    h = _mm(C[lc, b], X[b]) + cbs[lc, b]
    xn = jnp.sin(30.0 * h).astype(jnp.bfloat16)
    X[b] = xn

    @pl.when(l == 3)
    def _():
        cw4 = jnp.concatenate([c4[b] for c4 in C4s], axis=0)
        o_ref[0] = _mm(cw4, xn) + cb4[b]


def _mega_call(coords_t, cw0, gs, cbs, cb4, w1r, w2r, w3r, w4r):
    g1, g2, g3, g4 = gs
    in_specs = [
        pl.BlockSpec((1, 2, _N), lambda s: (jnp.clip(s, 0, 15), 0, 0)),
        pl.BlockSpec((_B, _HID, 2), lambda s: (0, 0, 0)),
        pl.BlockSpec(g1.shape, lambda s: (0, 0)),
        pl.BlockSpec(g2.shape, lambda s: (0, 0)),
        pl.BlockSpec(g3.shape, lambda s: (0, 0)),
        pl.BlockSpec(g4.shape, lambda s: (0, 0)),
        pl.BlockSpec(cbs.shape, lambda s: (0, 0, 0, 0)),
        pl.BlockSpec(cb4.shape, lambda s: (0, 0, 0)),
        pl.BlockSpec((_ES[1], 16384), lambda s: (0, jnp.clip(s, 0, 3))),
        pl.BlockSpec((_ES[2], 4096), lambda s: (0, jnp.clip(s, 0, 15))),
        pl.BlockSpec((_ES[3], 4096),
                     lambda s: (0, jnp.clip((s - 8) // 2, 0, 15))),
        pl.BlockSpec((_ES[4], _HID), lambda s: (0, jnp.clip(s - 40, 0, 2))),
    ]
    return pl.pallas_call(
        _mega_body,
        grid=(4 * _B,),
        in_specs=in_specs,
        out_specs=pl.BlockSpec((1, _OUT, _N),
                               lambda s: (jnp.clip(s - 48, 0, 15), 0, 0)),
        out_shape=jax.ShapeDtypeStruct((_B, _OUT, _N), jnp.float32),
        scratch_shapes=[
            pltpu.VMEM((_B, _HID, _N), jnp.bfloat16),
            pltpu.VMEM((4, _B, _HID, _HID), jnp.bfloat16),
            pltpu.VMEM((_B, 1, _HID), jnp.bfloat16),
            pltpu.VMEM((_B, 1, _HID), jnp.bfloat16),
            pltpu.VMEM((_B, 1, _HID), jnp.bfloat16),
        ],
    )(coords_t, cw0, g1, g2, g3, g4, cbs, cb4, w1r, w2r, w3r, w4r)


def kernel(latents, coords, Wg0, Wg1, Wg2, Wg3, Wg4,
           bg0, bg1, bg2, bg3, bg4,
           W0, W1, W2, W3, W4, b0, b1, b2, b3, b4):
    wgs = tuple(w.T for w in (Wg0, Wg1, Wg2, Wg3, Wg4))
    bgs = tuple(b.reshape(1, -1) for b in (bg0, bg1, bg2, bg3, bg4))
    outs = [_HID, _HID, _HID, _HID, _OUT]
    bps = tuple(b.reshape(e, o) for b, e, o in
                zip((b0, b1, b2, b3, b4), _ES, outs))
    w0r = W0.reshape(_ES[0], _HID * 2)

    (g1, g2, g3, g4, cw0f, cb0, cb1, cb2, cb3, cb4) = _small_call(
        latents, wgs, bgs, w0r, bps)

    out_t = _mega_call(
        coords.transpose(0, 2, 1),
        cw0f.reshape(_B, _HID, 2),
        (g1, g2, g3, g4),
        jnp.stack([cb1, cb2, cb3, cb0]).reshape(4, _B, _HID, 1),
        cb4.reshape(_B, _OUT, 1),
        W1.reshape(_ES[1], _HID * _HID),
        W2.reshape(_ES[2], _HID * _HID),
        W3.reshape(_ES[3], _HID * _HID),
        W4.reshape(_ES[4], _OUT * _HID),
    )
    return out_t.transpose(0, 2, 1)


# speedup vs baseline: 1.1313x; 1.0385x over previous
import jax
import jax.numpy as jnp
from jax.experimental import pallas as pl
from jax.experimental.pallas import tpu as pltpu

_ES = [8, 16, 64, 256, 1024]
_HID = 256
_B = 16
_N = 1024
_LAT = 64
_OUT = 3


def _mm(a, b):
    return jax.lax.dot_general(
        a.astype(jnp.bfloat16), b.astype(jnp.bfloat16),
        (((1,), (0,)), ((), ())),
        preferred_element_type=jnp.float32)


def _small_body(lat_ref, wg0, wg1, wg2, wg3, wg4, bg0, bg1, bg2, bg3, bg4,
                w0r, bp0, bp1, bp2, bp3, bp4,
                g1o, g2o, g3o, g4o, cw0o, cb0o, cb1o, cb2o, cb3o, cb4o):
    wgs = (wg0, wg1, wg2, wg3, wg4)
    bgs = (bg0, bg1, bg2, bg3, bg4)
    bps = (bp0, bp1, bp2, bp3, bp4)
    gouts = (None, g1o, g2o, g3o, g4o)
    cbouts = (cb0o, cb1o, cb2o, cb3o, cb4o)
    for l in range(5):
        lat = lat_ref[:, l, :]
        g = _mm(lat, wgs[l][...]) + bgs[l][...]
        if l == 0:
            cw0o[...] = _mm(g, w0r[...])
        else:
            gouts[l][...] = g
        cbouts[l][...] = _mm(g, bps[l][...])


def _small_call(latents, wgs, bgs, w0r, bps):
    outs = [
        jax.ShapeDtypeStruct((_B, _ES[1]), jnp.float32),
        jax.ShapeDtypeStruct((_B, _ES[2]), jnp.float32),
        jax.ShapeDtypeStruct((_B, _ES[3]), jnp.float32),
        jax.ShapeDtypeStruct((_B, _ES[4]), jnp.float32),
        jax.ShapeDtypeStruct((_B, _HID * 2), jnp.float32),
        jax.ShapeDtypeStruct((_B, _HID), jnp.float32),
        jax.ShapeDtypeStruct((_B, _HID), jnp.float32),
        jax.ShapeDtypeStruct((_B, _HID), jnp.float32),
        jax.ShapeDtypeStruct((_B, _HID), jnp.float32),
        jax.ShapeDtypeStruct((_B, _OUT), jnp.float32),
    ]
    return pl.pallas_call(_small_body, out_shape=outs)(
        latents, *wgs, *bgs, w0r, *bps)


def _mega_body(ct, cw0, g1, g2, g3, g4, cbs, cb4,
               w1, w2, w3a, w3b, w4, o_ref, X, C, C4a, C4b, C4c):
    C4s = (C4a, C4b, C4c)
    s = pl.program_id(0)
    l = s // 4
    grp = s % 4

    @pl.when(s < 4)
    def _():
        v = _mm(g1[...], w1[...])
        C[0, :, pl.ds(s * 64, 64), :] = (
            v.reshape(_B, 64, _HID).astype(jnp.bfloat16))

    @pl.when(s < 8)
    def _():
        v = _mm(g2[...], w2[...])
        C[1, :, pl.ds(s * 32, 32), :] = (
            v.reshape(_B, 32, _HID).astype(jnp.bfloat16))

    @pl.when((s >= 2) & (s < 10))
    def _():
        c = s - 2
        va = _mm(g3[...], w3a[...])
        C[2, :, pl.ds(c * 32, 16), :] = (
            va.reshape(_B, 16, _HID).astype(jnp.bfloat16))
        vb = _mm(g3[...], w3b[...])
        C[2, :, pl.ds(c * 32 + 16, 16), :] = (
            vb.reshape(_B, 16, _HID).astype(jnp.bfloat16))

    for c, C4c in enumerate(C4s):
        @pl.when(s == 9 + c)
        def _(c=c, C4c=C4c):
            v = _mm(g4[...], w4[...])
            C4c[...] = v.reshape(_B, 1, _HID).astype(jnp.bfloat16)

    @pl.when(s == 0)
    def _():
        C[3] = jnp.concatenate(
            [cw0[...].astype(jnp.bfloat16),
             jnp.zeros((_B, _HID, _HID - 2), jnp.bfloat16)], axis=2)

    @pl.when(l == 0)
    def _():
        for i in range(4):
            X[grp * 4 + i] = jnp.concatenate(
                [ct[i].astype(jnp.bfloat16),
                 jnp.zeros((_HID - 2, _N), jnp.bfloat16)], axis=0)

    lc = jnp.where(l == 0, 3, l - 1)
    xns = []
    for i in range(4):
        b = grp * 4 + i
        h = _mm(C[lc, b], X[b]) + cbs[lc, b]
        xn = jnp.sin(30.0 * h).astype(jnp.bfloat16)
        X[b] = xn
        xns.append(xn)

    @pl.when(l == 3)
    def _():
        for i in range(4):
            b = grp * 4 + i
            cw4 = jnp.concatenate([c4[b] for c4 in C4s], axis=0)
            o_ref[i] = _mm(cw4, xns[i]) + cb4[b]


def _mega_call(coords_t, cw0, gs, cbs, cb4, w1r, w2r, w3r, w4r):
    g1, g2, g3, g4 = gs
    in_specs = [
        pl.BlockSpec((4, 2, _N), lambda s: (jnp.clip(s, 0, 3), 0, 0)),
        pl.BlockSpec((_B, _HID, 2), lambda s: (0, 0, 0)),
        pl.BlockSpec(g1.shape, lambda s: (0, 0)),
        pl.BlockSpec(g2.shape, lambda s: (0, 0)),
        pl.BlockSpec(g3.shape, lambda s: (0, 0)),
        pl.BlockSpec(g4.shape, lambda s: (0, 0)),
        pl.BlockSpec(cbs.shape, lambda s: (0, 0, 0, 0)),
        pl.BlockSpec(cb4.shape, lambda s: (0, 0, 0)),
        pl.BlockSpec((_ES[1], 16384), lambda s: (0, jnp.clip(s, 0, 3))),
        pl.BlockSpec((_ES[2], 8192), lambda s: (0, jnp.clip(s, 0, 7))),
        pl.BlockSpec((_ES[3], 4096),
                     lambda s: (0, 2 * jnp.clip(s - 2, 0, 7))),
        pl.BlockSpec((_ES[3], 4096),
                     lambda s: (0, 2 * jnp.clip(s - 2, 0, 7) + 1)),
        pl.BlockSpec((_ES[4], _HID), lambda s: (0, jnp.clip(s - 9, 0, 2))),
    ]
    return pl.pallas_call(
        _mega_body,
        grid=(16,),
        in_specs=in_specs,
        out_specs=pl.BlockSpec((4, _OUT, _N),
                               lambda s: (jnp.clip(s - 12, 0, 3), 0, 0)),
        out_shape=jax.ShapeDtypeStruct((_B, _OUT, _N), jnp.float32),
        scratch_shapes=[
            pltpu.VMEM((_B, _HID, _N), jnp.bfloat16),
            pltpu.VMEM((4, _B, _HID, _HID), jnp.bfloat16),
            pltpu.VMEM((_B, 1, _HID), jnp.bfloat16),
            pltpu.VMEM((_B, 1, _HID), jnp.bfloat16),
            pltpu.VMEM((_B, 1, _HID), jnp.bfloat16),
        ],
    )(coords_t, cw0, g1, g2, g3, g4, cbs, cb4, w1r, w2r, w3r, w3r, w4r)


def kernel(latents, coords, Wg0, Wg1, Wg2, Wg3, Wg4,
           bg0, bg1, bg2, bg3, bg4,
           W0, W1, W2, W3, W4, b0, b1, b2, b3, b4):
    wgs = tuple(w.T for w in (Wg0, Wg1, Wg2, Wg3, Wg4))
    bgs = tuple(b.reshape(1, -1) for b in (bg0, bg1, bg2, bg3, bg4))
    outs = [_HID, _HID, _HID, _HID, _OUT]
    bps = tuple(b.reshape(e, o) for b, e, o in
                zip((b0, b1, b2, b3, b4), _ES, outs))
    w0r = W0.reshape(_ES[0], _HID * 2)

    (g1, g2, g3, g4, cw0f, cb0, cb1, cb2, cb3, cb4) = _small_call(
        latents, wgs, bgs, w0r, bps)

    out_t = _mega_call(
        coords.transpose(0, 2, 1),
        cw0f.reshape(_B, _HID, 2),
        (g1, g2, g3, g4),
        jnp.stack([cb1, cb2, cb3, cb0]).reshape(4, _B, _HID, 1),
        cb4.reshape(_B, _OUT, 1),
        W1.reshape(_ES[1], _HID * _HID),
        W2.reshape(_ES[2], _HID * _HID),
        W3.reshape(_ES[3], _HID * _HID),
        W4.reshape(_ES[4], _OUT * _HID),
    )
    return out_t.transpose(0, 2, 1)


# speedup vs baseline: 1.1342x; 1.0026x over previous
import jax
import jax.numpy as jnp
from jax.experimental import pallas as pl
from jax.experimental.pallas import tpu as pltpu

_ES = [8, 16, 64, 256, 1024]
_HID = 256
_B = 16
_N = 1024
_LAT = 64
_OUT = 3


def _mm(a, b):
    return jax.lax.dot_general(
        a.astype(jnp.bfloat16), b.astype(jnp.bfloat16),
        (((1,), (0,)), ((), ())),
        preferred_element_type=jnp.float32)


def _small_body(lat_ref, wg0, wg1, wg2, wg3, wg4, bg0, bg1, bg2, bg3, bg4,
                w0r, bp0, bp1, bp2, bp3, bp4,
                g1o, g2o, g3o, g4o, cw0o, cb0o, cb1o, cb2o, cb3o, cb4o):
    wgs = (wg0, wg1, wg2, wg3, wg4)
    bgs = (bg0, bg1, bg2, bg3, bg4)
    bps = (bp0, bp1, bp2, bp3, bp4)
    gouts = (None, g1o, g2o, g3o, g4o)
    cbouts = (cb0o, cb1o, cb2o, cb3o, cb4o)
    for l in range(5):
        lat = lat_ref[:, l, :]
        g = _mm(lat, wgs[l][...]) + bgs[l][...]
        if l == 0:
            cw0o[...] = _mm(g, w0r[...])
        else:
            gouts[l][...] = g
        cbouts[l][...] = _mm(g, bps[l][...])


def _small_call(latents, wgs, bgs, w0r, bps):
    outs = [
        jax.ShapeDtypeStruct((_B, _ES[1]), jnp.float32),
        jax.ShapeDtypeStruct((_B, _ES[2]), jnp.float32),
        jax.ShapeDtypeStruct((_B, _ES[3]), jnp.float32),
        jax.ShapeDtypeStruct((_B, _ES[4]), jnp.float32),
        jax.ShapeDtypeStruct((_B, _HID * 2), jnp.float32),
        jax.ShapeDtypeStruct((_B, _HID), jnp.float32),
        jax.ShapeDtypeStruct((_B, _HID), jnp.float32),
        jax.ShapeDtypeStruct((_B, _HID), jnp.float32),
        jax.ShapeDtypeStruct((_B, _HID), jnp.float32),
        jax.ShapeDtypeStruct((_B, _OUT), jnp.float32),
    ]
    return pl.pallas_call(_small_body, out_shape=outs)(
        latents, *wgs, *bgs, w0r, *bps)


def _mega_body(ct, cw0, g1, g2, g3, g4, cbs, cb4,
               w1, w2, w3, w4, o_ref, X, C, C4a, C4b, C4c):
    C4s = (C4a, C4b, C4c)
    s = pl.program_id(0)
    l = s // 4
    grp = s % 4

    @pl.when(s < 4)
    def _():
        v = _mm(g1[...], w1[...])
        C[0, :, pl.ds(s * 64, 64), :] = (
            v.reshape(_B, 64, _HID).astype(jnp.bfloat16))

    @pl.when(s < 8)
    def _():
        v = _mm(g2[...], w2[...])
        C[1, :, pl.ds(s * 32, 32), :] = (
            v.reshape(_B, 32, _HID).astype(jnp.bfloat16))

    @pl.when((s >= 2) & (s < 10))
    def _():
        c = s - 2
        v = _mm(g3[...], w3[...])
        C[2, :, pl.ds(c * 32, 32), :] = (
            v.reshape(_B, 32, _HID).astype(jnp.bfloat16))

    for c, C4c in enumerate(C4s):
        @pl.when(s == 9 + c)
        def _(c=c, C4c=C4c):
            v = _mm(g4[...], w4[...])
            C4c[...] = v.reshape(_B, 1, _HID).astype(jnp.bfloat16)

    @pl.when(s == 0)
    def _():
        C[3] = jnp.concatenate(
            [cw0[...].astype(jnp.bfloat16),
             jnp.zeros((_B, _HID, _HID - 2), jnp.bfloat16)], axis=2)

    @pl.when(l == 0)
    def _():
        for i in range(4):
            X[grp * 4 + i] = jnp.concatenate(
                [ct[i].astype(jnp.bfloat16),
                 jnp.zeros((_HID - 2, _N), jnp.bfloat16)], axis=0)

    lc = jnp.where(l == 0, 3, l - 1)
    xns = []
    for i in range(4):
        b = grp * 4 + i
        h = _mm(C[lc, b], X[b]) + cbs[lc, b]
        xn = jnp.sin(30.0 * h).astype(jnp.bfloat16)
        X[b] = xn
        xns.append(xn)

    @pl.when(l == 3)
    def _():
        for i in range(4):
            b = grp * 4 + i
            cw4 = jnp.concatenate([c4[b] for c4 in C4s], axis=0)
            o_ref[i] = _mm(cw4, xns[i]) + cb4[b]


def _mega_call(coords_t, cw0, gs, cbs, cb4, w1r, w2r, w3r, w4r):
    g1, g2, g3, g4 = gs
    in_specs = [
        pl.BlockSpec((4, 2, _N), lambda s: (jnp.clip(s, 0, 3), 0, 0)),
        pl.BlockSpec((_B, _HID, 2), lambda s: (0, 0, 0)),
        pl.BlockSpec(g1.shape, lambda s: (0, 0)),
        pl.BlockSpec(g2.shape, lambda s: (0, 0)),
        pl.BlockSpec(g3.shape, lambda s: (0, 0)),
        pl.BlockSpec(g4.shape, lambda s: (0, 0)),
        pl.BlockSpec(cbs.shape, lambda s: (0, 0, 0, 0)),
        pl.BlockSpec(cb4.shape, lambda s: (0, 0, 0)),
        pl.BlockSpec((_ES[1], 16384), lambda s: (0, jnp.clip(s, 0, 3))),
        pl.BlockSpec((_ES[2], 8192), lambda s: (0, jnp.clip(s, 0, 7))),
        pl.BlockSpec((_ES[3], 8192), lambda s: (0, jnp.clip(s - 2, 0, 7))),
        pl.BlockSpec((_ES[4], _HID), lambda s: (0, jnp.clip(s - 9, 0, 2))),
    ]
    return pl.pallas_call(
        _mega_body,
        grid=(16,),
        in_specs=in_specs,
        out_specs=pl.BlockSpec((4, _OUT, _N),
                               lambda s: (jnp.clip(s - 12, 0, 3), 0, 0)),
        out_shape=jax.ShapeDtypeStruct((_B, _OUT, _N), jnp.float32),
        scratch_shapes=[
            pltpu.VMEM((_B, _HID, _N), jnp.bfloat16),
            pltpu.VMEM((4, _B, _HID, _HID), jnp.bfloat16),
            pltpu.VMEM((_B, 1, _HID), jnp.bfloat16),
            pltpu.VMEM((_B, 1, _HID), jnp.bfloat16),
            pltpu.VMEM((_B, 1, _HID), jnp.bfloat16),
        ],
    )(coords_t, cw0, g1, g2, g3, g4, cbs, cb4, w1r, w2r, w3r, w4r)


def kernel(latents, coords, Wg0, Wg1, Wg2, Wg3, Wg4,
           bg0, bg1, bg2, bg3, bg4,
           W0, W1, W2, W3, W4, b0, b1, b2, b3, b4):
    wgs = tuple(w.T for w in (Wg0, Wg1, Wg2, Wg3, Wg4))
    bgs = tuple(b.reshape(1, -1) for b in (bg0, bg1, bg2, bg3, bg4))
    outs = [_HID, _HID, _HID, _HID, _OUT]
    bps = tuple(b.reshape(e, o) for b, e, o in
                zip((b0, b1, b2, b3, b4), _ES, outs))
    w0r = W0.reshape(_ES[0], _HID * 2)

    (g1, g2, g3, g4, cw0f, cb0, cb1, cb2, cb3, cb4) = _small_call(
        latents, wgs, bgs, w0r, bps)

    out_t = _mega_call(
        coords.transpose(0, 2, 1),
        cw0f.reshape(_B, _HID, 2),
        (g1, g2, g3, g4),
        jnp.stack([cb1, cb2, cb3, cb0]).reshape(4, _B, _HID, 1),
        cb4.reshape(_B, _OUT, 1),
        W1.reshape(_ES[1], _HID * _HID),
        W2.reshape(_ES[2], _HID * _HID),
        W3.reshape(_ES[3], _HID * _HID),
        W4.reshape(_ES[4], _OUT * _HID),
    )
    return out_t.transpose(0, 2, 1)


# speedup vs baseline: 1.2834x; 1.1315x over previous
import jax
import jax.numpy as jnp
from jax.experimental import pallas as pl
from jax.experimental.pallas import tpu as pltpu

_ES = [8, 16, 64, 256, 1024]
_HID = 256
_B = 16
_N = 1024
_LAT = 64
_OUT = 3


def _mm(a, b):
    return jax.lax.dot_general(
        a.astype(jnp.bfloat16), b.astype(jnp.bfloat16),
        (((1,), (0,)), ((), ())),
        preferred_element_type=jnp.float32)


def _small_body(lat_ref, wg0, wg1, wg2, wg3, wg4, bg0, bg1, bg2, bg3, bg4,
                w0r, bp0, bp1, bp2, bp3, bp4,
                g1o, g2o, g3o, g4o, cw0o, cb0o, cb1o, cb2o, cb3o, cb4o):
    wgs = (wg0, wg1, wg2, wg3, wg4)
    bgs = (bg0, bg1, bg2, bg3, bg4)
    bps = (bp0, bp1, bp2, bp3, bp4)
    gouts = (None, g1o, g2o, g3o, g4o)
    cbouts = (cb0o, cb1o, cb2o, cb3o, cb4o)
    for l in range(5):
        lat = lat_ref[:, l, :]
        g = _mm(lat, wgs[l][...]) + bgs[l][...]
        if l == 0:
            cw0o[...] = _mm(g, w0r[...])
        else:
            gouts[l][...] = g
        cbouts[l][...] = _mm(g, bps[l][...])


def _small_call(latents, wgs, bgs, w0r, bps):
    outs = [
        jax.ShapeDtypeStruct((_B, _ES[1]), jnp.float32),
        jax.ShapeDtypeStruct((_B, _ES[2]), jnp.float32),
        jax.ShapeDtypeStruct((_B, _ES[3]), jnp.float32),
        jax.ShapeDtypeStruct((_B, _ES[4]), jnp.float32),
        jax.ShapeDtypeStruct((_B, _HID * 2), jnp.float32),
        jax.ShapeDtypeStruct((_B, _HID), jnp.float32),
        jax.ShapeDtypeStruct((_B, _HID), jnp.float32),
        jax.ShapeDtypeStruct((_B, _HID), jnp.float32),
        jax.ShapeDtypeStruct((_B, _HID), jnp.float32),
        jax.ShapeDtypeStruct((_B, _OUT), jnp.float32),
    ]
    return pl.pallas_call(_small_body, out_shape=outs)(
        latents, *wgs, *bgs, w0r, *bps)


def _mega_body(ct, cw0, g1, g2, g3, g4, cbs, cb4,
               w1, w2, w3, w4, o_ref, X, C, C4a, C4b, C4c):
    C4s = (C4a, C4b, C4c)
    s = pl.program_id(0)
    l = s // 4
    grp = s % 4

    @pl.when(s < 4)
    def _():
        v = _mm(g1[...], w1[...])
        C[0, :, pl.ds(s * 64, 64), :] = (
            v.reshape(_B, 64, _HID).astype(jnp.bfloat16))

    @pl.when(s < 8)
    def _():
        v = _mm(g2[...], w2[...])
        C[1, :, pl.ds(s * 32, 32), :] = (
            v.reshape(_B, 32, _HID).astype(jnp.bfloat16))

    @pl.when((s >= 2) & (s < 10))
    def _():
        c = s - 2
        v = _mm(g3[...], w3[...].reshape(_ES[3], 32 * _HID))
        C[2, :, pl.ds(c * 32, 32), :] = (
            v.reshape(_B, 32, _HID).astype(jnp.bfloat16))

    for c, C4c in enumerate(C4s):
        @pl.when(s == 9 + c)
        def _(c=c, C4c=C4c):
            v = _mm(g4[...], w4[...])
            C4c[...] = v.reshape(_B, 1, _HID).astype(jnp.bfloat16)

    @pl.when(s == 0)
    def _():
        C[3] = jnp.concatenate(
            [cw0[...].astype(jnp.bfloat16),
             jnp.zeros((_B, _HID, _HID - 2), jnp.bfloat16)], axis=2)

    @pl.when(l == 0)
    def _():
        for i in range(4):
            X[grp * 4 + i] = jnp.concatenate(
                [ct[i].astype(jnp.bfloat16),
                 jnp.zeros((_HID - 2, _N), jnp.bfloat16)], axis=0)

    lc = jnp.where(l == 0, 3, l - 1)
    xns = []
    for i in range(4):
        b = grp * 4 + i
        h = _mm(C[lc, b], X[b]) + cbs[lc, b]
        xn = jnp.sin(30.0 * h).astype(jnp.bfloat16)
        X[b] = xn
        xns.append(xn)

    @pl.when(l == 3)
    def _():
        for i in range(4):
            b = grp * 4 + i
            cw4 = jnp.concatenate([c4[b] for c4 in C4s], axis=0)
            o_ref[i] = _mm(cw4, xns[i]) + cb4[b]


def _mega_call(coords_t, cw0, gs, cbs, cb4, w1r, w2r, w3r, w4r):
    g1, g2, g3, g4 = gs
    in_specs = [
        pl.BlockSpec((4, 2, _N), lambda s: (jnp.clip(s, 0, 3), 0, 0)),
        pl.BlockSpec((_B, _HID, 2), lambda s: (0, 0, 0)),
        pl.BlockSpec(g1.shape, lambda s: (0, 0)),
        pl.BlockSpec(g2.shape, lambda s: (0, 0)),
        pl.BlockSpec(g3.shape, lambda s: (0, 0)),
        pl.BlockSpec(g4.shape, lambda s: (0, 0)),
        pl.BlockSpec(cbs.shape, lambda s: (0, 0, 0, 0)),
        pl.BlockSpec(cb4.shape, lambda s: (0, 0, 0)),
        pl.BlockSpec((_ES[1], 16384), lambda s: (0, jnp.clip(s, 0, 3))),
        pl.BlockSpec((_ES[2], 8192), lambda s: (0, jnp.clip(s, 0, 7))),
        pl.BlockSpec((_ES[3], 32, _HID),
                     lambda s: (0, jnp.clip(s - 2, 0, 7), 0)),
        pl.BlockSpec((_ES[4], _HID), lambda s: (0, jnp.clip(s - 9, 0, 2))),
    ]
    return pl.pallas_call(
        _mega_body,
        grid=(16,),
        in_specs=in_specs,
        out_specs=pl.BlockSpec((4, _OUT, _N),
                               lambda s: (jnp.clip(s - 12, 0, 3), 0, 0)),
        out_shape=jax.ShapeDtypeStruct((_B, _OUT, _N), jnp.float32),
        scratch_shapes=[
            pltpu.VMEM((_B, _HID, _N), jnp.bfloat16),
            pltpu.VMEM((4, _B, _HID, _HID), jnp.bfloat16),
            pltpu.VMEM((_B, 1, _HID), jnp.bfloat16),
            pltpu.VMEM((_B, 1, _HID), jnp.bfloat16),
            pltpu.VMEM((_B, 1, _HID), jnp.bfloat16),
        ],
    )(coords_t, cw0, g1, g2, g3, g4, cbs, cb4, w1r, w2r, w3r, w4r)


def kernel(latents, coords, Wg0, Wg1, Wg2, Wg3, Wg4,
           bg0, bg1, bg2, bg3, bg4,
           W0, W1, W2, W3, W4, b0, b1, b2, b3, b4):
    wgs = tuple(w.T for w in (Wg0, Wg1, Wg2, Wg3, Wg4))
    bgs = tuple(b.reshape(1, -1) for b in (bg0, bg1, bg2, bg3, bg4))
    outs = [_HID, _HID, _HID, _HID, _OUT]
    bps = tuple(b.reshape(e, o) for b, e, o in
                zip((b0, b1, b2, b3, b4), _ES, outs))
    w0r = W0.reshape(_ES[0], _HID * 2)

    (g1, g2, g3, g4, cw0f, cb0, cb1, cb2, cb3, cb4) = _small_call(
        latents, wgs, bgs, w0r, bps)

    out_t = _mega_call(
        coords.transpose(0, 2, 1),
        cw0f.reshape(_B, _HID, 2),
        (g1, g2, g3, g4),
        jnp.stack([cb1, cb2, cb3, cb0]).reshape(4, _B, _HID, 1),
        cb4.reshape(_B, _OUT, 1),
        W1.reshape(_ES[1], _HID * _HID),
        W2.reshape(_ES[2], _HID * _HID),
        W3.reshape(_ES[3], _HID, _HID),
        W4.reshape(_ES[4], _OUT * _HID),
    )
    return out_t.transpose(0, 2, 1)


# speedup vs baseline: 1.3443x; 1.0474x over previous
import jax
import jax.numpy as jnp
from jax.experimental import pallas as pl
from jax.experimental.pallas import tpu as pltpu

_ES = [8, 16, 64, 256, 1024]
_HID = 256
_B = 16
_N = 1024
_LAT = 64
_OUT = 3


def _mm(a, b):
    return jax.lax.dot_general(
        a.astype(jnp.bfloat16), b.astype(jnp.bfloat16),
        (((1,), (0,)), ((), ())),
        preferred_element_type=jnp.float32)


def _small_body(lat_ref, wg0, wg1, wg2, wg3, wg4, bg0, bg1, bg2, bg3, bg4,
                w0r, w4r, bp0, bp1, bp2, bp3, bp4,
                g1o, g2o, g3o, cw0o, cw4o, cb0o, cb1o, cb2o, cb3o, cb4o):
    wgs = (wg0, wg1, wg2, wg3, wg4)
    bgs = (bg0, bg1, bg2, bg3, bg4)
    bps = (bp0, bp1, bp2, bp3, bp4)
    gouts = (None, g1o, g2o, g3o, None)
    cbouts = (cb0o, cb1o, cb2o, cb3o, cb4o)
    for l in range(5):
        lat = lat_ref[:, l, :]
        g = _mm(lat, wgs[l][...]) + bgs[l][...]
        if l == 0:
            cw0o[...] = _mm(g, w0r[...])
        elif l == 4:
            cw4o[...] = _mm(g, w4r[...].reshape(_ES[4], _OUT * _HID))
        else:
            gouts[l][...] = g
        cbouts[l][...] = _mm(g, bps[l][...])


def _small_call(latents, wgs, bgs, w0r, w4r, bps):
    outs = [
        jax.ShapeDtypeStruct((_B, _ES[1]), jnp.float32),
        jax.ShapeDtypeStruct((_B, _ES[2]), jnp.float32),
        jax.ShapeDtypeStruct((_B, _ES[3]), jnp.float32),
        jax.ShapeDtypeStruct((_B, _HID * 2), jnp.float32),
        jax.ShapeDtypeStruct((_B, _OUT * _HID), jnp.float32),
        jax.ShapeDtypeStruct((_B, _HID), jnp.float32),
        jax.ShapeDtypeStruct((_B, _HID), jnp.float32),
        jax.ShapeDtypeStruct((_B, _HID), jnp.float32),
        jax.ShapeDtypeStruct((_B, _HID), jnp.float32),
        jax.ShapeDtypeStruct((_B, _OUT), jnp.float32),
    ]
    return pl.pallas_call(_small_body, out_shape=outs)(
        latents, *wgs, *bgs, w0r, w4r, *bps)


def _mega_body(ct, cw0, cw4, g1, g2, g3, cbs, cb4,
               w1, w2, w3, o_ref, X, C):
    s = pl.program_id(0)
    l = s // 4
    grp = s % 4

    @pl.when(s < 4)
    def _():
        v = _mm(g1[...], w1[...].reshape(_ES[1], 64 * _HID))
        C[0, :, pl.ds(s * 64, 64), :] = (
            v.reshape(_B, 64, _HID).astype(jnp.bfloat16))

    @pl.when(s < 8)
    def _():
        v = _mm(g2[...], w2[...].reshape(_ES[2], 32 * _HID))
        C[1, :, pl.ds(s * 32, 32), :] = (
            v.reshape(_B, 32, _HID).astype(jnp.bfloat16))

    @pl.when((s >= 2) & (s < 10))
    def _():
        c = s - 2
        v = _mm(g3[...], w3[...].reshape(_ES[3], 32 * _HID))
        C[2, :, pl.ds(c * 32, 32), :] = (
            v.reshape(_B, 32, _HID).astype(jnp.bfloat16))

    @pl.when(s == 0)
    def _():
        C[3] = jnp.concatenate(
            [cw0[...].astype(jnp.bfloat16),
             jnp.zeros((_B, _HID, _HID - 2), jnp.bfloat16)], axis=2)

    @pl.when(l == 0)
    def _():
        for i in range(4):
            X[grp * 4 + i] = jnp.concatenate(
                [ct[i].astype(jnp.bfloat16),
                 jnp.zeros((_HID - 2, _N), jnp.bfloat16)], axis=0)

    lc = jnp.where(l == 0, 3, l - 1)
    xns = []
    for i in range(4):
        b = grp * 4 + i
        h = _mm(C[lc, b], X[b]) + cbs[lc, b]
        xn = jnp.sin(30.0 * h).astype(jnp.bfloat16)
        X[b] = xn
        xns.append(xn)

    @pl.when(l == 3)
    def _():
        for i in range(4):
            b = grp * 4 + i
            o_ref[i] = _mm(cw4[b], xns[i]) + cb4[b]


def _mega_call(coords_t, cw0, cw4, gs, cbs, cb4, w1r, w2r, w3r):
    g1, g2, g3 = gs
    in_specs = [
        pl.BlockSpec((4, 2, _N), lambda s: (jnp.clip(s, 0, 3), 0, 0)),
        pl.BlockSpec((_B, _HID, 2), lambda s: (0, 0, 0)),
        pl.BlockSpec((_B, _OUT, _HID), lambda s: (0, 0, 0)),
        pl.BlockSpec(g1.shape, lambda s: (0, 0)),
        pl.BlockSpec(g2.shape, lambda s: (0, 0)),
        pl.BlockSpec(g3.shape, lambda s: (0, 0)),
        pl.BlockSpec(cbs.shape, lambda s: (0, 0, 0, 0)),
        pl.BlockSpec(cb4.shape, lambda s: (0, 0, 0)),
        pl.BlockSpec((_ES[1], 64, _HID), lambda s: (0, jnp.clip(s, 0, 3), 0)),
        pl.BlockSpec((_ES[2], 32, _HID), lambda s: (0, jnp.clip(s, 0, 7), 0)),
        pl.BlockSpec((_ES[3], 32, _HID),
                     lambda s: (0, jnp.clip(s - 2, 0, 7), 0)),
    ]
    return pl.pallas_call(
        _mega_body,
        grid=(16,),
        in_specs=in_specs,
        out_specs=pl.BlockSpec((4, _OUT, _N),
                               lambda s: (jnp.clip(s - 12, 0, 3), 0, 0)),
        out_shape=jax.ShapeDtypeStruct((_B, _OUT, _N), jnp.float32),
        scratch_shapes=[
            pltpu.VMEM((_B, _HID, _N), jnp.bfloat16),
            pltpu.VMEM((4, _B, _HID, _HID), jnp.bfloat16),
        ],
    )(coords_t, cw0, cw4, g1, g2, g3, cbs, cb4, w1r, w2r, w3r)


def kernel(latents, coords, Wg0, Wg1, Wg2, Wg3, Wg4,
           bg0, bg1, bg2, bg3, bg4,
           W0, W1, W2, W3, W4, b0, b1, b2, b3, b4):
    wgs = tuple(w.T for w in (Wg0, Wg1, Wg2, Wg3, Wg4))
    bgs = tuple(b.reshape(1, -1) for b in (bg0, bg1, bg2, bg3, bg4))
    outs = [_HID, _HID, _HID, _HID, _OUT]
    bps = tuple(b.reshape(e, o) for b, e, o in
                zip((b0, b1, b2, b3, b4), _ES, outs))
    w0r = W0.reshape(_ES[0], _HID * 2)

    (g1, g2, g3, cw0f, cw4f, cb0, cb1, cb2, cb3, cb4) = _small_call(
        latents, wgs, bgs, w0r, W4.reshape(_ES[4], _OUT, _HID), bps)

    out_t = _mega_call(
        coords.transpose(0, 2, 1),
        cw0f.reshape(_B, _HID, 2),
        cw4f.reshape(_B, _OUT, _HID),
        (g1, g2, g3),
        jnp.stack([cb1, cb2, cb3, cb0]).reshape(4, _B, _HID, 1),
        cb4.reshape(_B, _OUT, 1),
        W1.reshape(_ES[1], _HID, _HID),
        W2.reshape(_ES[2], _HID, _HID),
        W3.reshape(_ES[3], _HID, _HID),
    )
    return out_t.transpose(0, 2, 1)
